# baseline (speedup 1.0000x reference)
import functools

import jax
import jax.numpy as jnp
import numpy as np
from jax import lax
from jax.experimental import pallas as pl
from jax.experimental.pallas import tpu as pltpu
from jax.experimental.pallas import tpu_sc as plsc

LOG2 = float(np.log(2.0))

_NC = 2
_NS = 16
_NW = _NC * _NS


def _ssp(x):
    return jax.nn.softplus(x) - LOG2


def _in2f_body(s_ref, w_ref, y_ref):
    y_ref[...] = jnp.dot(s_ref[...], w_ref[...],
                         preferred_element_type=jnp.float32)


def _in2f(s2d, Win2f):
    A, D = s2d.shape
    F = Win2f.shape[1]
    blk = 2000
    return pl.pallas_call(
        _in2f_body,
        grid=(A // blk,),
        in_specs=[
            pl.BlockSpec((blk, D), lambda i: (i, 0)),
            pl.BlockSpec((D, F), lambda i: (0, 0)),
        ],
        out_specs=pl.BlockSpec((blk, F), lambda i: (i, 0)),
        out_shape=jax.ShapeDtypeStruct((A, F), jnp.float32),
    )(s2d, Win2f)


def _sc_gather(y, idx3):
    A, F = y.shape
    NW, nch, ch = idx3.shape
    per_w = nch * ch
    E = NW * per_w
    k = 5
    ng = nch // k
    RB = k * ch

    mesh = plsc.VectorSubcoreMesh(core_axis_name="c", subcore_axis_name="s")

    @functools.partial(
        pl.kernel,
        mesh=mesh,
        out_type=jax.ShapeDtypeStruct((E, F), jnp.float32),
        scratch_types=[
            pltpu.VMEM((nch, ch), jnp.int32),
            pltpu.VMEM((RB, F), jnp.float32),
            pltpu.VMEM((RB, F), jnp.float32),
            pltpu.SemaphoreType.DMA,
            pltpu.SemaphoreType.DMA,
        ],
    )
    def gather_kernel(y_hbm, idx_hbm, out_hbm, idx_v, rows0, rows1, s0, s1):
        wid = lax.axis_index("s") * _NC + lax.axis_index("c")
        base = wid * per_w
        pltpu.sync_copy(idx_hbm.at[wid], idx_v)

        def fire(g, rows, sem):
            for b in range(k):
                pltpu.async_copy(y_hbm.at[idx_v.at[g * k + b]],
                                 rows.at[pl.ds(b * ch, ch)], sem)

        def drain(rows, sem):
            pltpu.make_async_copy(out_hbm.at[pl.ds(0, RB), :], rows,
                                  sem).wait()

        def scatter(g, rows):
            off = pl.multiple_of(base + g * RB, 8)
            pltpu.sync_copy(rows, out_hbm.at[pl.ds(off, RB), :])

        fire(0, rows0, s0)

        def body(i, carry):
            ga = 1 + 2 * i
            gb = 2 + 2 * i
            fire(ga, rows1, s1)
            drain(rows0, s0)
            scatter(ga - 1, rows0)
            fire(gb, rows0, s0)
            drain(rows1, s1)
            scatter(ga, rows1)
            return carry

        lax.fori_loop(0, (ng - 1) // 2, body, 0)
        drain(rows0, s0)
        scatter(ng - 1, rows0)

    return gather_kernel(y, idx3)


def _fused_body(N, AB, fij_ref, ynb_ref, mask_ref, W1_ref, b1_ref, W2_ref,
                b2_ref, Wf2out_ref, bf2out_ref, Wd_ref, bd_ref, out_ref):
    F = fij_ref.shape[1]
    h = jnp.dot(fij_ref[...], W1_ref[...],
                preferred_element_type=jnp.float32) + b1_ref[...]
    w = jnp.dot(_ssp(h), W2_ref[...],
                preferred_element_type=jnp.float32) + b2_ref[...]
    w = w * mask_ref[...]
    p = w * ynb_ref[...]
    agg = jnp.sum(p.reshape(AB, N, F), axis=1)
    t = _ssp(jnp.dot(agg, Wf2out_ref[...],
                     preferred_element_type=jnp.float32) + bf2out_ref[...])
    out_ref[...] = jnp.dot(t, Wd_ref[...],
                           preferred_element_type=jnp.float32) + bd_ref[...]


def _fused(fij2d, ynb, mask2d, N, W1, b1, W2, b2, Wf2out, bf2out, Wd, bd,
           blk0=0, nblk=None):
    E, S = fij2d.shape
    F = W1.shape[1]
    D = Wf2out.shape[1]
    AB = 200
    EB = AB * N
    if nblk is None:
        nblk = E // N // AB
    wspec = pl.BlockSpec((S, F), lambda i: (0, 0))
    bspec = pl.BlockSpec((1, F), lambda i: (0, 0))
    return pl.pallas_call(
        functools.partial(_fused_body, N, AB),
        grid=(nblk,),
        in_specs=[
            pl.BlockSpec((EB, S), lambda i: (blk0 + i, 0)),
            pl.BlockSpec((EB, F), lambda i: (i, 0)),
            pl.BlockSpec((EB, 1), lambda i: (blk0 + i, 0)),
            wspec, bspec, wspec, bspec,
            pl.BlockSpec((F, D), lambda i: (0, 0)),
            pl.BlockSpec((1, D), lambda i: (0, 0)),
            pl.BlockSpec((D, D), lambda i: (0, 0)),
            pl.BlockSpec((1, D), lambda i: (0, 0)),
        ],
        out_specs=pl.BlockSpec((AB, D), lambda i: (i, 0)),
        out_shape=jax.ShapeDtypeStruct((nblk * AB, D), jnp.float32),
    )(fij2d, ynb, mask2d, W1, b1, W2, b2, Wf2out, bf2out, Wd, bd)


def kernel(s, neighbor_mask, neighbors, f_ij, W1, b1, W2, b2, Win2f,
           Wf2out, bf2out, Wd, bd):
    B, A, N = neighbors.shape
    D = s.shape[-1]
    S = f_ij.shape[-1]
    F = Win2f.shape[1]
    E = B * A * N

    s2d = s.reshape(B * A, D)
    idx = neighbors.reshape(E).astype(jnp.int32)
    fij2d = f_ij.reshape(E, S)
    mask2d = neighbor_mask.reshape(E, 1)

    y = _in2f(s2d, Win2f)

    nh = 2
    Eh = E // nh
    per_w = Eh // _NW
    ch = 40
    nblk_h = Eh // N // 200
    outs = []
    ynbs = [
        _sc_gather(y, lax.slice_in_dim(idx, h * Eh, (h + 1) * Eh)
                   .reshape(_NW, per_w // ch, ch))
        for h in range(nh)
    ]
    for h in range(nh):
        outs.append(_fused(
            fij2d, ynbs[h], mask2d, N,
            W1, b1.reshape(1, F), W2, b2.reshape(1, F),
            Wf2out, bf2out.reshape(1, D), Wd, bd.reshape(1, D),
            blk0=h * nblk_h, nblk=nblk_h))
    out = jnp.concatenate(outs, axis=0)
    return out.reshape(B, A, D)


# speedup vs baseline: 1.0482x; 1.0482x over previous
import functools

import jax
import jax.numpy as jnp
import numpy as np
from jax import lax
from jax.experimental import pallas as pl
from jax.experimental.pallas import tpu as pltpu
from jax.experimental.pallas import tpu_sc as plsc

LOG2 = float(np.log(2.0))

_NC = 2
_NS = 16
_NW = _NC * _NS


def _ssp(x):
    return jax.nn.softplus(x) - LOG2


def _in2f_body(s_ref, w_ref, y_ref):
    y_ref[...] = jnp.dot(s_ref[...], w_ref[...],
                         preferred_element_type=jnp.float32)


def _in2f(s2d, Win2f):
    A, D = s2d.shape
    F = Win2f.shape[1]
    blk = 2000
    return pl.pallas_call(
        _in2f_body,
        grid=(A // blk,),
        in_specs=[
            pl.BlockSpec((blk, D), lambda i: (i, 0)),
            pl.BlockSpec((D, F), lambda i: (0, 0)),
        ],
        out_specs=pl.BlockSpec((blk, F), lambda i: (i, 0)),
        out_shape=jax.ShapeDtypeStruct((A, F), jnp.float32),
    )(s2d, Win2f)


def _sc_gather(y, idx3):
    A, F = y.shape
    NW, nch, ch = idx3.shape
    per_w = nch * ch
    E = NW * per_w
    nbuf = 5
    ng = nch // nbuf

    mesh = plsc.VectorSubcoreMesh(core_axis_name="c", subcore_axis_name="s")

    @functools.partial(
        pl.kernel,
        mesh=mesh,
        out_type=jax.ShapeDtypeStruct((E, F), jnp.float32),
        scratch_types=[
            pltpu.VMEM((nch, ch), jnp.int32),
            pltpu.VMEM((nbuf, ch, F), jnp.float32),
        ]
        + [pltpu.SemaphoreType.DMA] * (2 * nbuf),
    )
    def gather_kernel(y_hbm, idx_hbm, out_hbm, idx_v, rows, *sems):
        gsem = sems[:nbuf]
        ssem = sems[nbuf:]
        wid = lax.axis_index("s") * _NC + lax.axis_index("c")
        base = wid * per_w
        pltpu.sync_copy(idx_hbm.at[wid], idx_v)

        def fire_gather(c, b):
            pltpu.async_copy(y_hbm.at[idx_v.at[c]], rows.at[b], gsem[b])

        def wait_gather(b):
            pltpu.make_async_copy(out_hbm.at[pl.ds(0, ch), :], rows.at[b],
                                  gsem[b]).wait()

        def fire_scatter(c, b):
            off = pl.multiple_of(base + c * ch, 8)
            pltpu.async_copy(rows.at[b], out_hbm.at[pl.ds(off, ch), :],
                             ssem[b])

        def wait_scatter(b):
            pltpu.make_async_copy(out_hbm.at[pl.ds(0, ch), :], rows.at[b],
                                  ssem[b]).wait()

        for b in range(nbuf):
            fire_gather(b, b)

        def body(i, carry):
            for b in range(nbuf):
                wait_gather(b)
                fire_scatter(nbuf * i + b, b)
            for b in range(nbuf):
                wait_scatter(b)
                fire_gather(nbuf * (i + 1) + b, b)
            return carry

        lax.fori_loop(0, ng - 1, body, 0)
        for b in range(nbuf):
            wait_gather(b)
            fire_scatter(nbuf * (ng - 1) + b, b)
        for b in range(nbuf):
            wait_scatter(b)

    return gather_kernel(y, idx3)


def _fused_body(N, AB, fij_ref, ynb_ref, mask_ref, W1_ref, b1_ref, W2_ref,
                b2_ref, Wf2out_ref, bf2out_ref, Wd_ref, bd_ref, out_ref):
    F = fij_ref.shape[1]
    h = jnp.dot(fij_ref[...], W1_ref[...],
                preferred_element_type=jnp.float32) + b1_ref[...]
    w = jnp.dot(_ssp(h), W2_ref[...],
                preferred_element_type=jnp.float32) + b2_ref[...]
    w = w * mask_ref[...]
    p = w * ynb_ref[...]
    agg = jnp.sum(p.reshape(AB, N, F), axis=1)
    t = _ssp(jnp.dot(agg, Wf2out_ref[...],
                     preferred_element_type=jnp.float32) + bf2out_ref[...])
    out_ref[...] = jnp.dot(t, Wd_ref[...],
                           preferred_element_type=jnp.float32) + bd_ref[...]


def _fused(fij2d, ynb, mask2d, N, W1, b1, W2, b2, Wf2out, bf2out, Wd, bd,
           blk0=0, nblk=None):
    E, S = fij2d.shape
    F = W1.shape[1]
    D = Wf2out.shape[1]
    AB = 200
    EB = AB * N
    if nblk is None:
        nblk = E // N // AB
    wspec = pl.BlockSpec((S, F), lambda i: (0, 0))
    bspec = pl.BlockSpec((1, F), lambda i: (0, 0))
    return pl.pallas_call(
        functools.partial(_fused_body, N, AB),
        grid=(nblk,),
        in_specs=[
            pl.BlockSpec((EB, S), lambda i: (blk0 + i, 0)),
            pl.BlockSpec((EB, F), lambda i: (i, 0)),
            pl.BlockSpec((EB, 1), lambda i: (blk0 + i, 0)),
            wspec, bspec, wspec, bspec,
            pl.BlockSpec((F, D), lambda i: (0, 0)),
            pl.BlockSpec((1, D), lambda i: (0, 0)),
            pl.BlockSpec((D, D), lambda i: (0, 0)),
            pl.BlockSpec((1, D), lambda i: (0, 0)),
        ],
        out_specs=pl.BlockSpec((AB, D), lambda i: (i, 0)),
        out_shape=jax.ShapeDtypeStruct((nblk * AB, D), jnp.float32),
    )(fij2d, ynb, mask2d, W1, b1, W2, b2, Wf2out, bf2out, Wd, bd)


def kernel(s, neighbor_mask, neighbors, f_ij, W1, b1, W2, b2, Win2f,
           Wf2out, bf2out, Wd, bd):
    B, A, N = neighbors.shape
    D = s.shape[-1]
    S = f_ij.shape[-1]
    F = Win2f.shape[1]
    E = B * A * N

    s2d = s.reshape(B * A, D)
    idx = neighbors.reshape(E).astype(jnp.int32)
    fij2d = f_ij.reshape(E, S)
    mask2d = neighbor_mask.reshape(E, 1)

    y = _in2f(s2d, Win2f)

    per_w = E // _NW
    ch = 80
    ynb = _sc_gather(y, idx.reshape(_NW, per_w // ch, ch))
    out = _fused(fij2d, ynb, mask2d, N,
                 W1, b1.reshape(1, F), W2, b2.reshape(1, F),
                 Wf2out, bf2out.reshape(1, D), Wd, bd.reshape(1, D))
    return out.reshape(B, A, D)


# speedup vs baseline: 1.3758x; 1.3125x over previous
import functools

import jax
import jax.numpy as jnp
import numpy as np
from jax import lax
from jax.experimental import pallas as pl
from jax.experimental.pallas import tpu as pltpu
from jax.experimental.pallas import tpu_sc as plsc

LOG2 = float(np.log(2.0))

_NC = 2
_NS = 16
_NW = _NC * _NS


def _ssp(x):
    return jax.nn.softplus(x) - LOG2


def _in2f_body(s_ref, w_ref, y_ref):
    y_ref[...] = jnp.dot(s_ref[...], w_ref[...],
                         preferred_element_type=jnp.float32)


def _in2f(s2d, Win2f):
    A, D = s2d.shape
    F = Win2f.shape[1]
    blk = 2000
    return pl.pallas_call(
        _in2f_body,
        grid=(A // blk,),
        in_specs=[
            pl.BlockSpec((blk, D), lambda i: (i, 0)),
            pl.BlockSpec((D, F), lambda i: (0, 0)),
        ],
        out_specs=pl.BlockSpec((blk, F), lambda i: (i, 0)),
        out_shape=jax.ShapeDtypeStruct((A, F), jnp.float32),
    )(s2d, Win2f)


def _sc_gather(y, idx3):
    A, F = y.shape
    NW, nch, ch = idx3.shape
    per_w = nch * ch
    E = NW * per_w
    nbuf = 5
    ng = nch // nbuf

    mesh = plsc.VectorSubcoreMesh(core_axis_name="c", subcore_axis_name="s")

    @functools.partial(
        pl.kernel,
        mesh=mesh,
        out_type=jax.ShapeDtypeStruct((E, F), jnp.float32),
        scratch_types=[
            pltpu.VMEM((nch, ch), jnp.int32),
            pltpu.VMEM((nbuf, ch, F), jnp.float32),
        ]
        + [pltpu.SemaphoreType.DMA] * (2 * nbuf),
    )
    def gather_kernel(y_hbm, idx_hbm, out_hbm, idx_v, rows, *sems):
        gsem = sems[:nbuf]
        ssem = sems[nbuf:]
        wid = lax.axis_index("s") * _NC + lax.axis_index("c")
        base = wid * per_w
        pltpu.sync_copy(idx_hbm.at[wid], idx_v)

        def fire_gather(c, b):
            pltpu.async_copy(y_hbm.at[idx_v.at[c]], rows.at[b], gsem[b])

        def wait_gather(b):
            pltpu.make_async_copy(out_hbm.at[pl.ds(0, ch), :], rows.at[b],
                                  gsem[b]).wait()

        def fire_scatter(c, b):
            off = pl.multiple_of(base + c * ch, 8)
            pltpu.async_copy(rows.at[b], out_hbm.at[pl.ds(off, ch), :],
                             ssem[b])

        def wait_scatter(b):
            pltpu.make_async_copy(out_hbm.at[pl.ds(0, ch), :], rows.at[b],
                                  ssem[b]).wait()

        for b in range(nbuf):
            fire_gather(b, b)

        def body(i, carry):
            for b in range(nbuf):
                wait_gather(b)
                fire_scatter(nbuf * i + b, b)
            for b in range(nbuf):
                wait_scatter(b)
                fire_gather(nbuf * (i + 1) + b, b)
            return carry

        lax.fori_loop(0, ng - 1, body, 0)
        for b in range(nbuf):
            wait_gather(b)
            fire_scatter(nbuf * (ng - 1) + b, b)
        for b in range(nbuf):
            wait_scatter(b)

    return gather_kernel(y, idx3)


def _fused_body(N, AB, fij_ref, ynb_ref, mask_ref, W1_ref, b1_ref, W2_ref,
                b2_ref, Wf2out_ref, bf2out_ref, Wd_ref, bd_ref, out_ref):
    F = fij_ref.shape[1]
    h = jnp.dot(fij_ref[...], W1_ref[...],
                preferred_element_type=jnp.float32) + b1_ref[...]
    w = jnp.dot(_ssp(h), W2_ref[...],
                preferred_element_type=jnp.float32) + b2_ref[...]
    p = w * ynb_ref[...]
    p3 = p.reshape(AB, N, F) * mask_ref[...][:, :, None]
    agg = jnp.sum(p3, axis=1)
    t = _ssp(jnp.dot(agg, Wf2out_ref[...],
                     preferred_element_type=jnp.float32) + bf2out_ref[...])
    out_ref[...] = jnp.dot(t, Wd_ref[...],
                           preferred_element_type=jnp.float32) + bd_ref[...]


def _fused(fij2d, ynb, maskAN, N, W1, b1, W2, b2, Wf2out, bf2out, Wd, bd,
           blk0=0, nblk=None):
    E, S = fij2d.shape
    F = W1.shape[1]
    D = Wf2out.shape[1]
    AB = 200
    EB = AB * N
    if nblk is None:
        nblk = E // N // AB
    wspec = pl.BlockSpec((S, F), lambda i: (0, 0))
    bspec = pl.BlockSpec((1, F), lambda i: (0, 0))
    return pl.pallas_call(
        functools.partial(_fused_body, N, AB),
        grid=(nblk,),
        in_specs=[
            pl.BlockSpec((EB, S), lambda i: (blk0 + i, 0)),
            pl.BlockSpec((EB, F), lambda i: (i, 0)),
            pl.BlockSpec((AB, N), lambda i: (blk0 + i, 0)),
            wspec, bspec, wspec, bspec,
            pl.BlockSpec((F, D), lambda i: (0, 0)),
            pl.BlockSpec((1, D), lambda i: (0, 0)),
            pl.BlockSpec((D, D), lambda i: (0, 0)),
            pl.BlockSpec((1, D), lambda i: (0, 0)),
        ],
        out_specs=pl.BlockSpec((AB, D), lambda i: (i, 0)),
        out_shape=jax.ShapeDtypeStruct((nblk * AB, D), jnp.float32),
    )(fij2d, ynb, maskAN, W1, b1, W2, b2, Wf2out, bf2out, Wd, bd)


def kernel(s, neighbor_mask, neighbors, f_ij, W1, b1, W2, b2, Win2f,
           Wf2out, bf2out, Wd, bd):
    B, A, N = neighbors.shape
    D = s.shape[-1]
    S = f_ij.shape[-1]
    F = Win2f.shape[1]
    E = B * A * N

    s2d = s.reshape(B * A, D)
    idx = neighbors.reshape(E).astype(jnp.int32)
    fij2d = f_ij.reshape(E, S)
    maskAN = neighbor_mask.reshape(B * A, N)

    y = _in2f(s2d, Win2f)

    per_w = E // _NW
    ch = 80
    ynb = _sc_gather(y, idx.reshape(_NW, per_w // ch, ch))
    out = _fused(fij2d, ynb, maskAN, N,
                 W1, b1.reshape(1, F), W2, b2.reshape(1, F),
                 Wf2out, bf2out.reshape(1, D), Wd, bd.reshape(1, D))
    return out.reshape(B, A, D)


# speedup vs baseline: 1.5017x; 1.0916x over previous
import functools

import jax
import jax.numpy as jnp
import numpy as np
from jax import lax
from jax.experimental import pallas as pl
from jax.experimental.pallas import tpu as pltpu
from jax.experimental.pallas import tpu_sc as plsc

LOG2 = float(np.log(2.0))

_NC = 2
_NS = 16
_NW = _NC * _NS


LOG2E = 1.4426950408889634


def _ssp(x):
    return LOG2 * (jnp.log2(1.0 + jnp.exp2(x * LOG2E)) - 1.0)


def _in2f_body(s_ref, w_ref, y_ref):
    y_ref[...] = jnp.dot(s_ref[...], w_ref[...],
                         preferred_element_type=jnp.float32)


def _in2f(s2d, Win2f):
    A, D = s2d.shape
    F = Win2f.shape[1]
    blk = 2000
    return pl.pallas_call(
        _in2f_body,
        grid=(A // blk,),
        in_specs=[
            pl.BlockSpec((blk, D), lambda i: (i, 0)),
            pl.BlockSpec((D, F), lambda i: (0, 0)),
        ],
        out_specs=pl.BlockSpec((blk, F), lambda i: (i, 0)),
        out_shape=jax.ShapeDtypeStruct((A, F), jnp.float32),
    )(s2d, Win2f)


def _sc_gather(y, idx3):
    A, F = y.shape
    NW, nch, ch = idx3.shape
    per_w = nch * ch
    E = NW * per_w
    nbuf = 5
    ng = nch // nbuf

    mesh = plsc.VectorSubcoreMesh(core_axis_name="c", subcore_axis_name="s")

    @functools.partial(
        pl.kernel,
        mesh=mesh,
        out_type=jax.ShapeDtypeStruct((E, F), jnp.float32),
        scratch_types=[
            pltpu.VMEM((nch, ch), jnp.int32),
            pltpu.VMEM((nbuf, ch, F), jnp.float32),
        ]
        + [pltpu.SemaphoreType.DMA] * (2 * nbuf),
    )
    def gather_kernel(y_hbm, idx_hbm, out_hbm, idx_v, rows, *sems):
        gsem = sems[:nbuf]
        ssem = sems[nbuf:]
        wid = lax.axis_index("s") * _NC + lax.axis_index("c")
        base = wid * per_w
        pltpu.sync_copy(idx_hbm.at[wid], idx_v)

        def fire_gather(c, b):
            pltpu.async_copy(y_hbm.at[idx_v.at[c]], rows.at[b], gsem[b])

        def wait_gather(b):
            pltpu.make_async_copy(out_hbm.at[pl.ds(0, ch), :], rows.at[b],
                                  gsem[b]).wait()

        def fire_scatter(c, b):
            off = pl.multiple_of(base + c * ch, 8)
            pltpu.async_copy(rows.at[b], out_hbm.at[pl.ds(off, ch), :],
                             ssem[b])

        def wait_scatter(b):
            pltpu.make_async_copy(out_hbm.at[pl.ds(0, ch), :], rows.at[b],
                                  ssem[b]).wait()

        for b in range(nbuf):
            fire_gather(b, b)

        def body(i, carry):
            for b in range(nbuf):
                wait_gather(b)
                fire_scatter(nbuf * i + b, b)
            for b in range(nbuf):
                wait_scatter(b)
                fire_gather(nbuf * (i + 1) + b, b)
            return carry

        lax.fori_loop(0, ng - 1, body, 0)
        for b in range(nbuf):
            wait_gather(b)
            fire_scatter(nbuf * (ng - 1) + b, b)
        for b in range(nbuf):
            wait_scatter(b)

    return gather_kernel(y, idx3)


def _fused_body(N, AB, fij_ref, ynb_ref, mask_ref, W1_ref, b1_ref, W2_ref,
                b2_ref, Wf2out_ref, bf2out_ref, Wd_ref, bd_ref, out_ref):
    F = fij_ref.shape[1]
    h = jnp.dot(fij_ref[...], W1_ref[...],
                preferred_element_type=jnp.float32) + b1_ref[...]
    w = jnp.dot(_ssp(h), W2_ref[...],
                preferred_element_type=jnp.float32) + b2_ref[...]
    p = w * ynb_ref[...]
    p3 = p.reshape(AB, N, F) * mask_ref[...][:, :, None]
    agg = jnp.sum(p3, axis=1)
    t = _ssp(jnp.dot(agg, Wf2out_ref[...],
                     preferred_element_type=jnp.float32) + bf2out_ref[...])
    out_ref[...] = jnp.dot(t, Wd_ref[...],
                           preferred_element_type=jnp.float32) + bd_ref[...]


def _fused(fij2d, ynb, maskAN, N, W1, b1, W2, b2, Wf2out, bf2out, Wd, bd,
           blk0=0, nblk=None):
    E, S = fij2d.shape
    F = W1.shape[1]
    D = Wf2out.shape[1]
    AB = 200
    EB = AB * N
    if nblk is None:
        nblk = E // N // AB
    wspec = pl.BlockSpec((S, F), lambda i: (0, 0))
    bspec = pl.BlockSpec((1, F), lambda i: (0, 0))
    return pl.pallas_call(
        functools.partial(_fused_body, N, AB),
        grid=(nblk,),
        in_specs=[
            pl.BlockSpec((EB, S), lambda i: (blk0 + i, 0)),
            pl.BlockSpec((EB, F), lambda i: (i, 0)),
            pl.BlockSpec((AB, N), lambda i: (blk0 + i, 0)),
            wspec, bspec, wspec, bspec,
            pl.BlockSpec((F, D), lambda i: (0, 0)),
            pl.BlockSpec((1, D), lambda i: (0, 0)),
            pl.BlockSpec((D, D), lambda i: (0, 0)),
            pl.BlockSpec((1, D), lambda i: (0, 0)),
        ],
        out_specs=pl.BlockSpec((AB, D), lambda i: (i, 0)),
        out_shape=jax.ShapeDtypeStruct((nblk * AB, D), jnp.float32),
    )(fij2d, ynb, maskAN, W1, b1, W2, b2, Wf2out, bf2out, Wd, bd)


def kernel(s, neighbor_mask, neighbors, f_ij, W1, b1, W2, b2, Win2f,
           Wf2out, bf2out, Wd, bd):
    B, A, N = neighbors.shape
    D = s.shape[-1]
    S = f_ij.shape[-1]
    F = Win2f.shape[1]
    E = B * A * N

    s2d = s.reshape(B * A, D)
    idx = neighbors.reshape(E).astype(jnp.int32)
    fij2d = f_ij.reshape(E, S)
    maskAN = neighbor_mask.reshape(B * A, N)

    y = _in2f(s2d, Win2f)

    per_w = E // _NW
    ch = 80
    ynb = _sc_gather(y, idx.reshape(_NW, per_w // ch, ch))
    out = _fused(fij2d, ynb, maskAN, N,
                 W1, b1.reshape(1, F), W2, b2.reshape(1, F),
                 Wf2out, bf2out.reshape(1, D), Wd, bd.reshape(1, D))
    return out.reshape(B, A, D)
